# Initial kernel scaffold; baseline (speedup 1.0000x reference)
import jax
import jax.numpy as jnp
from jax.experimental import pallas as pl


def kernel(x, edge_index, edge_attr, batch, W_ih_n, W_hh_n, b_ih_n, b_hh_n, W_ih_e, W_hh_e, b_ih_e, b_hh_e, W1a, b1a, gma, bta, W2a, b2a, W1b, b1b, gmb, btb, W2b, b2b, W_lin, b_lin):
    raise NotImplementedError("write your pallas kernel here")


# speedup vs baseline: 2.4215x; 2.4215x over previous
import functools

import jax
import jax.numpy as jnp
from jax import lax
from jax.experimental import pallas as pl
from jax.experimental.pallas import tpu as pltpu
from jax.experimental.pallas import tpu_sc as plsc

_N, _E, _H, _NG = 10000, 320000, 128, 16
_NB = 2000
_EB = 8000
_C = 80
_TPT = _E // 16
_NCH = _TPT // _C
_NR = 2560
_NT = 4 * _NR
_NPH = _NR + 128
_ZR = _NPH // 16


def _lstm_body(x_ref, w_ref, b_ref, o_ref):
    gates = jnp.dot(x_ref[...], w_ref[...],
                    preferred_element_type=jnp.float32) + b_ref[...]
    i = gates[:, :_H]
    g = gates[:, _H:]
    o_ref[...] = jax.nn.sigmoid(i) * jnp.tanh(g)


def _lstm(x, w_ig, b_ig, rows, blk):
    fin = x.shape[1]
    return pl.pallas_call(
        _lstm_body,
        grid=(rows // blk,),
        in_specs=[pl.BlockSpec((blk, fin), lambda i: (i, 0)),
                  pl.BlockSpec((fin, 2 * _H), lambda i: (0, 0)),
                  pl.BlockSpec((1, 2 * _H), lambda i: (0, 0))],
        out_specs=pl.BlockSpec((blk, _H), lambda i: (i, 0)),
        out_shape=jax.ShapeDtypeStruct((rows, _H), jnp.float32),
    )(x, w_ig, b_ig)


@functools.cache
def _sc_conv_kernel():
    mesh = plsc.VectorSubcoreMesh(core_axis_name="c", subcore_axis_name="s")

    @functools.partial(
        pl.kernel,
        mesh=mesh,
        out_type=(jax.ShapeDtypeStruct((_NT, _H), jnp.float32),
                  jax.ShapeDtypeStruct((_NT, _H), jnp.float32)),
        scratch_types=[
            pltpu.VMEM_SHARED((_NPH, _H), jnp.float32),
            pltpu.VMEM_SHARED((_NPH, _H), jnp.float32),
            pltpu.VMEM((_C,), jnp.int32),
            pltpu.VMEM((_C,), jnp.int32),
            pltpu.VMEM((_C, _H), jnp.float32),
            pltpu.VMEM((_C, _H), jnp.float32),
            pltpu.VMEM((_C, _H), jnp.float32),
            pltpu.VMEM((_C, _H), jnp.float32),
            pltpu.VMEM((_ZR, _H), jnp.float32),
            pltpu.SemaphoreType.DMA,
            pltpu.SemaphoreType.DMA,
        ],
    )
    def body(x_hbm, ea_hbm, src_hbm, dst_hbm, s1_hbm, s2_hbm,
             s1_acc, s2_acc, srcv, dstv, xv, eav, wv, mv, zbuf, sem1, sem2):
        c = lax.axis_index("c")
        t = lax.axis_index("s")
        zero = jnp.zeros((16,), jnp.float32)

        def zrow(r, carry):
            for j in range(_H // 16):
                zbuf[r, pl.ds(16 * j, 16)] = zero
            return carry
        lax.fori_loop(0, _ZR, zrow, 0, unroll=2)

        def do_round(r, carry):
            noff = (2 * r + c) * _NR
            pltpu.sync_copy(zbuf, s1_acc.at[pl.ds(t * _ZR, _ZR)])
            pltpu.sync_copy(zbuf, s2_acc.at[pl.ds(t * _ZR, _ZR)])
            plsc.subcore_barrier()

            def chunk(k, carry2):
                base = t * _TPT + k * _C
                pltpu.sync_copy(src_hbm.at[pl.ds(base, _C)], srcv)
                pltpu.sync_copy(dst_hbm.at[pl.ds(base, _C)], dstv)

                gcp = pltpu.async_copy(x_hbm.at[srcv], xv, sem1)
                ecp = pltpu.async_copy(ea_hbm.at[pl.ds(base, _C)], eav, sem2)

                def loc(j, carry3):
                    sl = pl.ds(16 * j, 16)
                    d = dstv[sl]
                    dl = d - noff
                    oob = (dl < 0) | (dl >= _NR)
                    dl = jnp.where(oob, _NR + jnp.bitwise_and(d, 127), dl)
                    dstv[sl] = dl
                    return carry3
                lax.fori_loop(0, _C // 16, loc, 0, unroll=_C // 16)

                gcp.wait()
                ecp.wait()

                def edge(e, carry3):
                    for j in range(_H // 16):
                        sl = pl.ds(16 * j, 16)
                        msg = jnp.maximum(xv[e, sl] + eav[e, sl], 0.0) + 1e-7
                        w = jnp.exp(msg)
                        wv[e, sl] = w
                        mv[e, sl] = msg * w
                    return carry3
                lax.fori_loop(0, _C, edge, 0)

                pltpu.sync_copy(wv, s1_acc.at[dstv], add=True)
                pltpu.sync_copy(mv, s2_acc.at[dstv], add=True)
                return carry2
            lax.fori_loop(0, _NCH, chunk, 0)

            plsc.subcore_barrier()
            wr = _NR // 16
            rs = pl.ds(t * wr, wr)
            os = pl.ds(noff + t * wr, wr)
            pltpu.sync_copy(s1_acc.at[rs], s1_hbm.at[os])
            pltpu.sync_copy(s2_acc.at[rs], s2_hbm.at[os])
            plsc.subcore_barrier()
            return carry
        lax.fori_loop(0, 2, do_round, 0)

    return body


def _sc_conv(x, ea, src, dst):
    return _sc_conv_kernel()(x, ea, src, dst)


def _mlp1_body(s1_ref, s2_ref, xin_ref, w_ref, b_ref, h1_ref, stats_ref,
               sacc, sqacc):
    i = pl.program_id(0)
    node = s2_ref[...] / (s1_ref[...] + 1e-16) + xin_ref[...]
    h1 = jnp.dot(node, w_ref[...], preferred_element_type=jnp.float32) \
        + b_ref[...]
    h1_ref[...] = h1

    @pl.when(i == 0)
    def _():
        sacc[...] = jnp.zeros_like(sacc)
        sqacc[...] = jnp.zeros_like(sqacc)
    sacc[...] += jnp.sum(h1, axis=0, keepdims=True)
    sqacc[...] += jnp.sum(h1 * h1, axis=0, keepdims=True)

    @pl.when(i == pl.num_programs(0) - 1)
    def _():
        stats_ref[...] = jnp.concatenate([sacc[...], sqacc[...]], axis=0)


def _mlp1(s1, s2, xin, w1t, b1):
    return pl.pallas_call(
        _mlp1_body,
        grid=(_N // _NB,),
        in_specs=[pl.BlockSpec((_NB, _H), lambda i: (i, 0)),
                  pl.BlockSpec((_NB, _H), lambda i: (i, 0)),
                  pl.BlockSpec((_NB, _H), lambda i: (i, 0)),
                  pl.BlockSpec((_H, 2 * _H), lambda i: (0, 0)),
                  pl.BlockSpec((1, 2 * _H), lambda i: (0, 0))],
        out_specs=[pl.BlockSpec((_NB, 2 * _H), lambda i: (i, 0)),
                   pl.BlockSpec((2, 2 * _H), lambda i: (0, 0))],
        out_shape=[jax.ShapeDtypeStruct((_N, 2 * _H), jnp.float32),
                   jax.ShapeDtypeStruct((2, 2 * _H), jnp.float32)],
        scratch_shapes=[pltpu.VMEM((1, 2 * _H), jnp.float32),
                        pltpu.VMEM((1, 2 * _H), jnp.float32)],
    )(s1, s2, xin, w1t, b1)


def _mlp2_body(h1_ref, stats_ref, gm_ref, bt_ref, w_ref, b_ref, bc_ref,
               y_ref, gs_ref, psum, psq, pcnt):
    i = pl.program_id(0)
    mu = stats_ref[0:1, :] * (1.0 / _N)
    var = stats_ref[1:2, :] * (1.0 / _N) - mu * mu
    h1n = (h1_ref[...] - mu) * lax.rsqrt(var + 1e-5) * gm_ref[...] \
        + bt_ref[...]
    h1n = jnp.maximum(h1n, 0.0)
    y = jnp.dot(h1n, w_ref[...], preferred_element_type=jnp.float32) \
        + b_ref[...]
    y = jnp.maximum(y, 0.0)
    y_ref[...] = y

    oh = (bc_ref[...] ==
          lax.broadcasted_iota(jnp.int32, (_NB, _NG), 1).astype(jnp.float32)
          ).astype(jnp.float32)
    dn = (((0,), (0,)), ((), ()))
    ps = lax.dot_general(oh, y, dn, preferred_element_type=jnp.float32)
    pq = lax.dot_general(oh, y * y, dn, preferred_element_type=jnp.float32)
    pc = lax.dot_general(oh, jnp.ones_like(y), dn,
                         preferred_element_type=jnp.float32)

    @pl.when(i == 0)
    def _():
        psum[...] = jnp.zeros_like(psum)
        psq[...] = jnp.zeros_like(psq)
        pcnt[...] = jnp.zeros_like(pcnt)
    psum[...] += ps
    psq[...] += pq
    pcnt[...] += pc

    @pl.when(i == pl.num_programs(0) - 1)
    def _():
        gs_ref[...] = jnp.concatenate([psum[...], psq[...], pcnt[...]],
                                      axis=0)


def _mlp2(h1, stats, gm, bt, w2t, b2, bc):
    return pl.pallas_call(
        _mlp2_body,
        grid=(_N // _NB,),
        in_specs=[pl.BlockSpec((_NB, 2 * _H), lambda i: (i, 0)),
                  pl.BlockSpec((2, 2 * _H), lambda i: (0, 0)),
                  pl.BlockSpec((1, 2 * _H), lambda i: (0, 0)),
                  pl.BlockSpec((1, 2 * _H), lambda i: (0, 0)),
                  pl.BlockSpec((2 * _H, _H), lambda i: (0, 0)),
                  pl.BlockSpec((1, _H), lambda i: (0, 0)),
                  pl.BlockSpec((_NB, 1), lambda i: (i, 0))],
        out_specs=[pl.BlockSpec((_NB, _H), lambda i: (i, 0)),
                   pl.BlockSpec((3 * _NG, _H), lambda i: (0, 0))],
        out_shape=[jax.ShapeDtypeStruct((_N, _H), jnp.float32),
                   jax.ShapeDtypeStruct((3 * _NG, _H), jnp.float32)],
        scratch_shapes=[pltpu.VMEM((_NG, _H), jnp.float32),
                        pltpu.VMEM((_NG, _H), jnp.float32),
                        pltpu.VMEM((_NG, _H), jnp.float32)],
    )(h1, stats, gm, bt, w2t, b2, bc)


def _inorm_body(y_ref, gs_ref, bc_ref, o_ref):
    cnt = jnp.maximum(gs_ref[2 * _NG:3 * _NG, :], 1.0)
    mean = gs_ref[0:_NG, :] / cnt
    var = gs_ref[_NG:2 * _NG, :] / cnt - mean * mean
    oh = (bc_ref[...] ==
          lax.broadcasted_iota(jnp.int32, (_NB, _NG), 1).astype(jnp.float32)
          ).astype(jnp.float32)
    rmean = jnp.dot(oh, mean, preferred_element_type=jnp.float32)
    rvar = jnp.dot(oh, var, preferred_element_type=jnp.float32)
    o_ref[...] = (y_ref[...] - rmean) * lax.rsqrt(rvar + 1e-5)


def _inorm(y, gs, bc):
    return pl.pallas_call(
        _inorm_body,
        grid=(_N // _NB,),
        in_specs=[pl.BlockSpec((_NB, _H), lambda i: (i, 0)),
                  pl.BlockSpec((3 * _NG, _H), lambda i: (0, 0)),
                  pl.BlockSpec((_NB, 1), lambda i: (i, 0))],
        out_specs=pl.BlockSpec((_NB, _H), lambda i: (i, 0)),
        out_shape=jax.ShapeDtypeStruct((_N, _H), jnp.float32),
    )(y, gs, bc)


def _pool_body(y_ref, gs_ref, bc_ref, wl_ref, bl_ref, o_ref, pool):
    i = pl.program_id(0)
    cnt = jnp.maximum(gs_ref[2 * _NG:3 * _NG, :], 1.0)
    mean = gs_ref[0:_NG, :] / cnt
    var = gs_ref[_NG:2 * _NG, :] / cnt - mean * mean
    oh = (bc_ref[...] ==
          lax.broadcasted_iota(jnp.int32, (_NB, _NG), 1).astype(jnp.float32)
          ).astype(jnp.float32)
    rmean = jnp.dot(oh, mean, preferred_element_type=jnp.float32)
    rvar = jnp.dot(oh, var, preferred_element_type=jnp.float32)
    out = (y_ref[...] - rmean) * lax.rsqrt(rvar + 1e-5)

    @pl.when(i == 0)
    def _():
        pool[...] = jnp.full((_NG, _H), -jnp.inf, jnp.float32)
    for g in range(_NG):
        mg = oh[:, g:g + 1]
        vals = jnp.where(mg > 0, out, -jnp.inf)
        pool[g:g + 1, :] = jnp.maximum(
            pool[g:g + 1, :], jnp.max(vals, axis=0, keepdims=True))

    @pl.when(i == pl.num_programs(0) - 1)
    def _():
        p = pool[...]
        p = jnp.where(p > -1e30, p, 0.0)
        z = jnp.dot(p, wl_ref[...], preferred_element_type=jnp.float32) \
            + bl_ref[...]
        o_ref[...] = jax.nn.sigmoid(z)


def _pool_final(y, gs, bc, wlt, bl):
    return pl.pallas_call(
        _pool_body,
        grid=(_N // _NB,),
        in_specs=[pl.BlockSpec((_NB, _H), lambda i: (i, 0)),
                  pl.BlockSpec((3 * _NG, _H), lambda i: (0, 0)),
                  pl.BlockSpec((_NB, 1), lambda i: (i, 0)),
                  pl.BlockSpec((_H, 1), lambda i: (0, 0)),
                  pl.BlockSpec((1, 1), lambda i: (0, 0))],
        out_specs=pl.BlockSpec((_NG, 1), lambda i: (0, 0)),
        out_shape=jax.ShapeDtypeStruct((_NG, 1), jnp.float32),
        scratch_shapes=[pltpu.VMEM((_NG, _H), jnp.float32)],
    )(y, gs, bc, wlt, bl)


def kernel(x, edge_index, edge_attr, batch, W_ih_n, W_hh_n, b_ih_n, b_hh_n,
           W_ih_e, W_hh_e, b_ih_e, b_hh_e, W1a, b1a, gma, bta, W2a, b2a,
           W1b, b1b, gmb, btb, W2b, b2b, W_lin, b_lin):
    src = edge_index[0]
    dst = edge_index[1]
    bn = b_ih_n + b_hh_n
    w_ig_n = jnp.concatenate([W_ih_n[:_H], W_ih_n[2 * _H:3 * _H]], axis=0).T
    b_ig_n = jnp.concatenate([bn[:_H], bn[2 * _H:3 * _H]])[None]
    be = b_ih_e + b_hh_e
    w_ig_e = jnp.concatenate([W_ih_e[:_H], W_ih_e[2 * _H:3 * _H]], axis=0).T
    b_ig_e = jnp.concatenate([be[:_H], be[2 * _H:3 * _H]])[None]
    bc = batch[:, None].astype(jnp.float32)

    xh = _lstm(x, w_ig_n, b_ig_n, _N, _NB)
    ea = _lstm(edge_attr, w_ig_e, b_ig_e, _E, _EB)

    s1, s2 = _sc_conv(xh, ea, src, dst)
    h1, st = _mlp1(s1, s2, xh, W1a.T, b1a[None])
    y, gs = _mlp2(h1, st, gma[None], bta[None], W2a.T, b2a[None], bc)
    h2 = _inorm(y, gs, bc)

    s1b, s2b = _sc_conv(h2, ea, src, dst)
    h1b, stb = _mlp1(s1b, s2b, h2, W1b.T, b1b[None])
    yb, gsb = _mlp2(h1b, stb, gmb[None], btb[None], W2b.T, b2b[None], bc)
    return _pool_final(yb, gsb, bc, W_lin.T, b_lin[None])


# speedup vs baseline: 3.3281x; 1.3744x over previous
import functools

import jax
import jax.numpy as jnp
from jax import lax
from jax.experimental import pallas as pl
from jax.experimental.pallas import tpu as pltpu
from jax.experimental.pallas import tpu_sc as plsc

_N, _E, _H, _NG = 10000, 320000, 128, 16
_NB = 2000
_EB = 8000
_C = 80
_TPT = _E // 16
_NCH = _TPT // _C
_NR = 2560
_NT = 4 * _NR
_NPH = _NR + 128
_ZR = _NPH // 16
_ZB = _ZR // 7


def _lstm_body(x_ref, w_ref, b_ref, o_ref):
    gates = jnp.dot(x_ref[...], w_ref[...],
                    preferred_element_type=jnp.float32) + b_ref[...]
    i = gates[:, :_H]
    g = gates[:, _H:]
    o_ref[...] = jax.nn.sigmoid(i) * jnp.tanh(g)


def _lstm(x, w_ig, b_ig, rows, blk):
    fin = x.shape[1]
    return pl.pallas_call(
        _lstm_body,
        grid=(rows // blk,),
        in_specs=[pl.BlockSpec((blk, fin), lambda i: (i, 0)),
                  pl.BlockSpec((fin, 2 * _H), lambda i: (0, 0)),
                  pl.BlockSpec((1, 2 * _H), lambda i: (0, 0))],
        out_specs=pl.BlockSpec((blk, _H), lambda i: (i, 0)),
        out_shape=jax.ShapeDtypeStruct((rows, _H), jnp.float32),
    )(x, w_ig, b_ig)


@functools.cache
def _sc_conv_kernel():
    mesh = plsc.VectorSubcoreMesh(core_axis_name="c", subcore_axis_name="s")

    @functools.partial(
        pl.kernel,
        mesh=mesh,
        out_type=(jax.ShapeDtypeStruct((_NT, _H), jnp.float32),
                  jax.ShapeDtypeStruct((_NT, _H), jnp.float32)),
        scratch_types=[
            pltpu.VMEM_SHARED((_NPH, _H), jnp.float32),
            pltpu.VMEM_SHARED((_NPH, _H), jnp.float32),
            pltpu.VMEM((_C,), jnp.int32),
            pltpu.VMEM((_C,), jnp.int32),
            pltpu.VMEM((_C,), jnp.int32),
            pltpu.VMEM((_C,), jnp.int32),
            pltpu.VMEM((_C, _H), jnp.float32),
            pltpu.VMEM((_C, _H), jnp.float32),
            pltpu.VMEM((_C, _H), jnp.float32),
            pltpu.VMEM((_C, _H), jnp.float32),
            pltpu.VMEM((_C, _H), jnp.float32),
            pltpu.VMEM((_C, _H), jnp.float32),
            pltpu.VMEM((_ZB, _H), jnp.float32),
            pltpu.SemaphoreType.DMA,
            pltpu.SemaphoreType.DMA,
            pltpu.SemaphoreType.DMA,
            pltpu.SemaphoreType.DMA,
        ],
    )
    def body(x_hbm, ea_hbm, src_hbm, dst_hbm, s1_hbm, s2_hbm,
             s1_acc, s2_acc, sv0, sv1, dv0, dv1,
             xv0, xv1, ev0, ev1, wv, mv, zbuf,
             sg0, sg1, se0, se1):
        c = lax.axis_index("c")
        t = lax.axis_index("s")
        zero = jnp.zeros((16,), jnp.float32)
        ebase = t * _TPT

        def zrow(r, carry):
            for j in range(_H // 16):
                zbuf[r, pl.ds(16 * j, 16)] = zero
            return carry
        lax.fori_loop(0, _ZB, zrow, 0, unroll=2)

        svs = (sv0, sv1)
        dvs = (dv0, dv1)
        xvs = (xv0, xv1)
        evs = (ev0, ev1)
        sgs = (sg0, sg1)
        ses = (se0, se1)

        def issue(kc, noff, b):
            off = kc * _C
            pltpu.sync_copy(src_hbm.at[pl.ds(ebase + off, _C)], svs[b])
            pltpu.sync_copy(dst_hbm.at[pl.ds(ebase + off, _C)], dvs[b])
            for j in range(_C // 16):
                sl = pl.ds(16 * j, 16)
                d = dvs[b][sl]
                dl = d - noff
                oob = (dl < 0) | (dl >= _NR)
                dvs[b][sl] = jnp.where(oob, _NR + jnp.bitwise_and(d, 127), dl)
            pltpu.async_copy(x_hbm.at[svs[b]], xvs[b], sgs[b])
            pltpu.async_copy(ea_hbm.at[pl.ds(ebase + off, _C)], evs[b], ses[b])

        def crunch(b):
            pltpu.make_async_copy(x_hbm.at[svs[b]], xvs[b], sgs[b]).wait()
            pltpu.make_async_copy(ea_hbm.at[pl.ds(0, _C)], evs[b],
                                  ses[b]).wait()

            def edge(e, carry3):
                for j in range(_H // 16):
                    sl = pl.ds(16 * j, 16)
                    msg = jnp.maximum(xvs[b][e, sl] + evs[b][e, sl], 0.0) \
                        + 1e-7
                    w = jnp.exp(msg)
                    wv[e, sl] = w
                    mv[e, sl] = msg * w
                return carry3
            lax.fori_loop(0, _C, edge, 0)

            pltpu.sync_copy(wv, s1_acc.at[dvs[b]], add=True)
            pltpu.sync_copy(mv, s2_acc.at[dvs[b]], add=True)

        def do_round(r, carry):
            noff = (2 * r + c) * _NR
            for p in range(_ZR // _ZB):
                pltpu.sync_copy(
                    zbuf, s1_acc.at[pl.ds(t * _ZR + p * _ZB, _ZB)])
                pltpu.sync_copy(
                    zbuf, s2_acc.at[pl.ds(t * _ZR + p * _ZB, _ZB)])
            plsc.subcore_barrier()

            issue(0, noff, 0)

            def pair(i, carry2):
                issue(2 * i + 1, noff, 1)
                crunch(0)
                nxt = 2 * i + 2
                nxt = jnp.where(nxt >= _NCH, 0, nxt)
                issue(nxt, noff, 0)
                crunch(1)
                return carry2
            lax.fori_loop(0, _NCH // 2, pair, 0)

            pltpu.make_async_copy(x_hbm.at[sv0], xv0, sg0).wait()
            pltpu.make_async_copy(ea_hbm.at[pl.ds(0, _C)], ev0, se0).wait()

            plsc.subcore_barrier()
            wr = _NR // 16
            rs = pl.ds(t * wr, wr)
            os = pl.ds(noff + t * wr, wr)
            pltpu.sync_copy(s1_acc.at[rs], s1_hbm.at[os])
            pltpu.sync_copy(s2_acc.at[rs], s2_hbm.at[os])
            plsc.subcore_barrier()
            return carry
        lax.fori_loop(0, 2, do_round, 0)

    return body

def _sc_conv(x, ea, src, dst):
    return _sc_conv_kernel()(x, ea, src, dst)


def _mlp1_body(s1_ref, s2_ref, xin_ref, w_ref, b_ref, h1_ref, stats_ref,
               sacc, sqacc):
    i = pl.program_id(0)
    node = s2_ref[...] / (s1_ref[...] + 1e-16) + xin_ref[...]
    h1 = jnp.dot(node, w_ref[...], preferred_element_type=jnp.float32) \
        + b_ref[...]
    h1_ref[...] = h1

    @pl.when(i == 0)
    def _():
        sacc[...] = jnp.zeros_like(sacc)
        sqacc[...] = jnp.zeros_like(sqacc)
    sacc[...] += jnp.sum(h1, axis=0, keepdims=True)
    sqacc[...] += jnp.sum(h1 * h1, axis=0, keepdims=True)

    @pl.when(i == pl.num_programs(0) - 1)
    def _():
        stats_ref[...] = jnp.concatenate([sacc[...], sqacc[...]], axis=0)


def _mlp1(s1, s2, xin, w1t, b1):
    return pl.pallas_call(
        _mlp1_body,
        grid=(_N // _NB,),
        in_specs=[pl.BlockSpec((_NB, _H), lambda i: (i, 0)),
                  pl.BlockSpec((_NB, _H), lambda i: (i, 0)),
                  pl.BlockSpec((_NB, _H), lambda i: (i, 0)),
                  pl.BlockSpec((_H, 2 * _H), lambda i: (0, 0)),
                  pl.BlockSpec((1, 2 * _H), lambda i: (0, 0))],
        out_specs=[pl.BlockSpec((_NB, 2 * _H), lambda i: (i, 0)),
                   pl.BlockSpec((2, 2 * _H), lambda i: (0, 0))],
        out_shape=[jax.ShapeDtypeStruct((_N, 2 * _H), jnp.float32),
                   jax.ShapeDtypeStruct((2, 2 * _H), jnp.float32)],
        scratch_shapes=[pltpu.VMEM((1, 2 * _H), jnp.float32),
                        pltpu.VMEM((1, 2 * _H), jnp.float32)],
    )(s1, s2, xin, w1t, b1)


def _mlp2_body(h1_ref, stats_ref, gm_ref, bt_ref, w_ref, b_ref, bc_ref,
               y_ref, gs_ref, psum, psq, pcnt):
    i = pl.program_id(0)
    mu = stats_ref[0:1, :] * (1.0 / _N)
    var = stats_ref[1:2, :] * (1.0 / _N) - mu * mu
    h1n = (h1_ref[...] - mu) * lax.rsqrt(var + 1e-5) * gm_ref[...] \
        + bt_ref[...]
    h1n = jnp.maximum(h1n, 0.0)
    y = jnp.dot(h1n, w_ref[...], preferred_element_type=jnp.float32) \
        + b_ref[...]
    y = jnp.maximum(y, 0.0)
    y_ref[...] = y

    oh = (bc_ref[...] ==
          lax.broadcasted_iota(jnp.int32, (_NB, _NG), 1).astype(jnp.float32)
          ).astype(jnp.float32)
    dn = (((0,), (0,)), ((), ()))
    ps = lax.dot_general(oh, y, dn, preferred_element_type=jnp.float32)
    pq = lax.dot_general(oh, y * y, dn, preferred_element_type=jnp.float32)
    pc = lax.dot_general(oh, jnp.ones_like(y), dn,
                         preferred_element_type=jnp.float32)

    @pl.when(i == 0)
    def _():
        psum[...] = jnp.zeros_like(psum)
        psq[...] = jnp.zeros_like(psq)
        pcnt[...] = jnp.zeros_like(pcnt)
    psum[...] += ps
    psq[...] += pq
    pcnt[...] += pc

    @pl.when(i == pl.num_programs(0) - 1)
    def _():
        gs_ref[...] = jnp.concatenate([psum[...], psq[...], pcnt[...]],
                                      axis=0)


def _mlp2(h1, stats, gm, bt, w2t, b2, bc):
    return pl.pallas_call(
        _mlp2_body,
        grid=(_N // _NB,),
        in_specs=[pl.BlockSpec((_NB, 2 * _H), lambda i: (i, 0)),
                  pl.BlockSpec((2, 2 * _H), lambda i: (0, 0)),
                  pl.BlockSpec((1, 2 * _H), lambda i: (0, 0)),
                  pl.BlockSpec((1, 2 * _H), lambda i: (0, 0)),
                  pl.BlockSpec((2 * _H, _H), lambda i: (0, 0)),
                  pl.BlockSpec((1, _H), lambda i: (0, 0)),
                  pl.BlockSpec((_NB, 1), lambda i: (i, 0))],
        out_specs=[pl.BlockSpec((_NB, _H), lambda i: (i, 0)),
                   pl.BlockSpec((3 * _NG, _H), lambda i: (0, 0))],
        out_shape=[jax.ShapeDtypeStruct((_N, _H), jnp.float32),
                   jax.ShapeDtypeStruct((3 * _NG, _H), jnp.float32)],
        scratch_shapes=[pltpu.VMEM((_NG, _H), jnp.float32),
                        pltpu.VMEM((_NG, _H), jnp.float32),
                        pltpu.VMEM((_NG, _H), jnp.float32)],
    )(h1, stats, gm, bt, w2t, b2, bc)


def _inorm_body(y_ref, gs_ref, bc_ref, o_ref):
    cnt = jnp.maximum(gs_ref[2 * _NG:3 * _NG, :], 1.0)
    mean = gs_ref[0:_NG, :] / cnt
    var = gs_ref[_NG:2 * _NG, :] / cnt - mean * mean
    oh = (bc_ref[...] ==
          lax.broadcasted_iota(jnp.int32, (_NB, _NG), 1).astype(jnp.float32)
          ).astype(jnp.float32)
    rmean = jnp.dot(oh, mean, preferred_element_type=jnp.float32)
    rvar = jnp.dot(oh, var, preferred_element_type=jnp.float32)
    o_ref[...] = (y_ref[...] - rmean) * lax.rsqrt(rvar + 1e-5)


def _inorm(y, gs, bc):
    return pl.pallas_call(
        _inorm_body,
        grid=(_N // _NB,),
        in_specs=[pl.BlockSpec((_NB, _H), lambda i: (i, 0)),
                  pl.BlockSpec((3 * _NG, _H), lambda i: (0, 0)),
                  pl.BlockSpec((_NB, 1), lambda i: (i, 0))],
        out_specs=pl.BlockSpec((_NB, _H), lambda i: (i, 0)),
        out_shape=jax.ShapeDtypeStruct((_N, _H), jnp.float32),
    )(y, gs, bc)


def _pool_body(y_ref, gs_ref, bc_ref, wl_ref, bl_ref, o_ref, pool):
    i = pl.program_id(0)
    cnt = jnp.maximum(gs_ref[2 * _NG:3 * _NG, :], 1.0)
    mean = gs_ref[0:_NG, :] / cnt
    var = gs_ref[_NG:2 * _NG, :] / cnt - mean * mean
    oh = (bc_ref[...] ==
          lax.broadcasted_iota(jnp.int32, (_NB, _NG), 1).astype(jnp.float32)
          ).astype(jnp.float32)
    rmean = jnp.dot(oh, mean, preferred_element_type=jnp.float32)
    rvar = jnp.dot(oh, var, preferred_element_type=jnp.float32)
    out = (y_ref[...] - rmean) * lax.rsqrt(rvar + 1e-5)

    @pl.when(i == 0)
    def _():
        pool[...] = jnp.full((_NG, _H), -jnp.inf, jnp.float32)
    for g in range(_NG):
        mg = oh[:, g:g + 1]
        vals = jnp.where(mg > 0, out, -jnp.inf)
        pool[g:g + 1, :] = jnp.maximum(
            pool[g:g + 1, :], jnp.max(vals, axis=0, keepdims=True))

    @pl.when(i == pl.num_programs(0) - 1)
    def _():
        p = pool[...]
        p = jnp.where(p > -1e30, p, 0.0)
        z = jnp.dot(p, wl_ref[...], preferred_element_type=jnp.float32) \
            + bl_ref[...]
        o_ref[...] = jax.nn.sigmoid(z)


def _pool_final(y, gs, bc, wlt, bl):
    return pl.pallas_call(
        _pool_body,
        grid=(_N // _NB,),
        in_specs=[pl.BlockSpec((_NB, _H), lambda i: (i, 0)),
                  pl.BlockSpec((3 * _NG, _H), lambda i: (0, 0)),
                  pl.BlockSpec((_NB, 1), lambda i: (i, 0)),
                  pl.BlockSpec((_H, 1), lambda i: (0, 0)),
                  pl.BlockSpec((1, 1), lambda i: (0, 0))],
        out_specs=pl.BlockSpec((_NG, 1), lambda i: (0, 0)),
        out_shape=jax.ShapeDtypeStruct((_NG, 1), jnp.float32),
        scratch_shapes=[pltpu.VMEM((_NG, _H), jnp.float32)],
    )(y, gs, bc, wlt, bl)


def kernel(x, edge_index, edge_attr, batch, W_ih_n, W_hh_n, b_ih_n, b_hh_n,
           W_ih_e, W_hh_e, b_ih_e, b_hh_e, W1a, b1a, gma, bta, W2a, b2a,
           W1b, b1b, gmb, btb, W2b, b2b, W_lin, b_lin):
    src = edge_index[0]
    dst = edge_index[1]
    bn = b_ih_n + b_hh_n
    w_ig_n = jnp.concatenate([W_ih_n[:_H], W_ih_n[2 * _H:3 * _H]], axis=0).T
    b_ig_n = jnp.concatenate([bn[:_H], bn[2 * _H:3 * _H]])[None]
    be = b_ih_e + b_hh_e
    w_ig_e = jnp.concatenate([W_ih_e[:_H], W_ih_e[2 * _H:3 * _H]], axis=0).T
    b_ig_e = jnp.concatenate([be[:_H], be[2 * _H:3 * _H]])[None]
    bc = batch[:, None].astype(jnp.float32)

    xh = _lstm(x, w_ig_n, b_ig_n, _N, _NB)
    ea = _lstm(edge_attr, w_ig_e, b_ig_e, _E, _EB)

    s1, s2 = _sc_conv(xh, ea, src, dst)
    h1, st = _mlp1(s1, s2, xh, W1a.T, b1a[None])
    y, gs = _mlp2(h1, st, gma[None], bta[None], W2a.T, b2a[None], bc)
    h2 = _inorm(y, gs, bc)

    s1b, s2b = _sc_conv(h2, ea, src, dst)
    h1b, stb = _mlp1(s1b, s2b, h2, W1b.T, b1b[None])
    yb, gsb = _mlp2(h1b, stb, gmb[None], btb[None], W2b.T, b2b[None], bc)
    return _pool_final(yb, gsb, bc, W_lin.T, b_lin[None])
